# baseline (speedup 1.0000x reference)
import jax
import jax.numpy as jnp
from jax import lax
from jax.experimental import pallas as pl
from jax.experimental.pallas import tpu as pltpu
from jax.experimental.pallas import tpu_sc as plsc

N = 10000
E = 160000
D = 256
HALF = D // 2
NS = 16
NC = 2
CK = 64
CH = 160
EPT = CH * CK
EP = NS * EPT
NBUF = 3
NPAD = 10240
RPT = NPAD // NS


def _sc_body(xcat, row_r, col2, zf, zc, ones_h,
             agg_out, cnt_out,
             rowv, colv, gbuf, onesv, s_agg, s_cnt, sem):
    cid = lax.axis_index("c")
    sid = lax.axis_index("s")

    pltpu.sync_copy(zf, s_agg.at[pl.ds(sid * RPT, RPT)])
    pltpu.sync_copy(zc, s_cnt.at[pl.ds(sid * RPT, RPT)])

    pltpu.sync_copy(row_r.at[sid], rowv)
    pltpu.sync_copy(col2.at[cid, sid], colv)
    pltpu.sync_copy(ones_h, onesv)

    plsc.subcore_barrier()

    def gather_start(j, p):
        pltpu.async_copy(xcat.at[colv.at[pl.ds(j * CK, CK)]], gbuf.at[p],
                         sem.at[p])

    def gather_wait(p):
        pltpu.make_async_copy(zf.at[pl.ds(0, CK)], gbuf.at[p],
                              sem.at[p]).wait()

    for b in range(NBUF):
        gather_start(b, b)

    def step(j, carry):
        p = lax.rem(j, NBUF)
        gather_wait(p)

        pltpu.sync_copy(gbuf.at[p], s_agg.at[rowv.at[pl.ds(j * CK, CK)]],
                        add=True)

        @pl.when(lax.rem(j, 2) == cid)
        def _():
            pltpu.sync_copy(onesv, s_cnt.at[rowv.at[pl.ds(j * CK, CK)]],
                            add=True)

        @pl.when(j < CH - NBUF)
        def _():
            gather_start(j + NBUF, p)
        return carry

    lax.fori_loop(0, CH, step, 0)

    plsc.subcore_barrier()

    pltpu.sync_copy(s_agg.at[pl.ds(sid * RPT, RPT)],
                    agg_out.at[cid, pl.ds(sid * RPT, RPT)])
    pltpu.sync_copy(s_cnt.at[pl.ds(sid * RPT, RPT)],
                    cnt_out.at[cid, pl.ds(sid * RPT, RPT)])


def _sc_aggregate(xcat, row_r, col2, zf, zc, ones_h):
    mesh = plsc.VectorSubcoreMesh(core_axis_name="c", subcore_axis_name="s")
    fn = pl.kernel(
        _sc_body,
        out_type=[
            jax.ShapeDtypeStruct((NC, NPAD, HALF), jnp.float32),
            jax.ShapeDtypeStruct((NC, NPAD), jnp.float32),
        ],
        mesh=mesh,
        scratch_types=[
            pltpu.VMEM((EPT,), jnp.int32),
            pltpu.VMEM((EPT,), jnp.int32),
            pltpu.VMEM((NBUF, CK, HALF), jnp.float32),
            pltpu.VMEM((CK,), jnp.float32),
            pltpu.VMEM_SHARED((NPAD, HALF), jnp.float32),
            pltpu.VMEM_SHARED((NPAD,), jnp.float32),
            pltpu.SemaphoreType.DMA((NBUF,)),
        ],
        name="sage_sc_aggregate",
    )
    return fn(xcat, row_r, col2, zf, zc, ones_h)


def _tc_body(x_ref, agg_ref, ca_ref, cb_ref, wx_ref, wa_ref, wb_ref, b_ref,
             o_ref):
    cinv = 1.0 / (ca_ref[...] + cb_ref[...] + 1e-6)
    y = jnp.dot(x_ref[...], wx_ref[...], preferred_element_type=jnp.float32)
    y = y + jnp.dot(agg_ref[0] * cinv, wa_ref[...],
                    preferred_element_type=jnp.float32)
    y = y + jnp.dot(agg_ref[1] * cinv, wb_ref[...],
                    preferred_element_type=jnp.float32)
    o_ref[...] = jnp.maximum(y + b_ref[...], 0.0)


def _tc_fuse(x, agg2, ca, cb, wx, wa, wb, b2):
    R = 400
    grid = (N // R,)
    return pl.pallas_call(
        _tc_body,
        grid=grid,
        in_specs=[
            pl.BlockSpec((R, D), lambda i: (i, 0)),
            pl.BlockSpec((NC, R, HALF), lambda i: (0, i, 0)),
            pl.BlockSpec((R, 1), lambda i: (i, 0)),
            pl.BlockSpec((R, 1), lambda i: (i, 0)),
            pl.BlockSpec((D, D), lambda i: (0, 0)),
            pl.BlockSpec((HALF, D), lambda i: (0, 0)),
            pl.BlockSpec((HALF, D), lambda i: (0, 0)),
            pl.BlockSpec((1, D), lambda i: (0, 0)),
        ],
        out_specs=pl.BlockSpec((R, D), lambda i: (i, 0)),
        out_shape=jax.ShapeDtypeStruct((N, D), jnp.float32),
        name="sage_tc_fuse",
    )(x, agg2, ca, cb, wx, wa, wb, b2)


def kernel(x, edge_index, W, b):
    row = edge_index[0].astype(jnp.int32)
    col = edge_index[1].astype(jnp.int32)
    pad = EP - E
    row = jnp.concatenate([row, jnp.full((pad,), NPAD - 1, jnp.int32)])
    col = jnp.concatenate([col, jnp.zeros((pad,), jnp.int32)])
    row_r = row.reshape(NS, EPT)
    col_r = col.reshape(NS, EPT)
    col2 = jnp.stack([2 * col_r, 2 * col_r + 1])
    xcat = x.reshape(2 * N, HALF)
    zf = jnp.zeros((RPT, HALF), jnp.float32)
    zc = jnp.zeros((RPT,), jnp.float32)
    ones_h = jnp.ones((CK,), jnp.float32)

    agg2, cpart = _sc_aggregate(xcat, row_r, col2, zf, zc, ones_h)

    ca = cpart[0][:N].reshape(N, 1)
    cb = cpart[1][:N].reshape(N, 1)
    Wt = W.T
    wx = Wt[:D]
    wa = Wt[D:D + HALF]
    wb = Wt[D + HALF:]
    return _tc_fuse(x, agg2, ca, cb, wx, wa, wb, b.reshape(1, D))


# speedup vs baseline: 1.8875x; 1.8875x over previous
import jax
import jax.numpy as jnp
from jax import lax
from jax.experimental import pallas as pl
from jax.experimental.pallas import tpu as pltpu
from jax.experimental.pallas import tpu_sc as plsc

N = 10000
E = 160000
D = 256
HALF = D // 2
NS = 16
NC = 2
EPT = E // NS
CK = 80
CH = EPT // CK
NPAD = 10240
RPT = NPAD // NS


def _sc_body(xcat, row_r, col2, zf, zc, ones_h,
             agg_out, cnt_out,
             rowv, colv, gbuf, onesv, s_agg, s_cnt, sem):
    cid = lax.axis_index("c")
    sid = lax.axis_index("s")

    pltpu.sync_copy(zf, s_agg.at[pl.ds(sid * RPT, RPT)])
    pltpu.sync_copy(zc, s_cnt.at[pl.ds(sid * RPT, RPT)])

    pltpu.sync_copy(row_r.at[sid], rowv)
    pltpu.sync_copy(col2.at[cid, sid], colv)
    pltpu.sync_copy(ones_h, onesv)

    plsc.subcore_barrier()

    def gather_start(j, p):
        pltpu.async_copy(xcat.at[colv.at[pl.ds(j * CK, CK)]], gbuf.at[p],
                         sem.at[p])

    def gather_wait(p):
        pltpu.make_async_copy(zf.at[pl.ds(0, CK)], gbuf.at[p],
                              sem.at[p]).wait()

    gather_start(0, 0)
    gather_start(1, 1)

    def step(j, carry):
        p = lax.rem(j, 2)
        gather_wait(p)

        pltpu.sync_copy(gbuf.at[p], s_agg.at[rowv.at[j]], add=True)

        @pl.when(p == cid)
        def _():
            pltpu.sync_copy(onesv, s_cnt.at[rowv.at[j]], add=True)

        @pl.when(j < CH - 2)
        def _():
            gather_start(j + 2, p)
        return carry

    lax.fori_loop(0, CH, step, 0)

    plsc.subcore_barrier()

    pltpu.sync_copy(s_agg.at[pl.ds(sid * RPT, RPT)],
                    agg_out.at[cid, pl.ds(sid * RPT, RPT)])
    pltpu.sync_copy(s_cnt.at[pl.ds(sid * RPT, RPT)],
                    cnt_out.at[cid, pl.ds(sid * RPT, RPT)])


def _sc_aggregate(xcat, row_r, col2, zf, zc, ones_h):
    mesh = plsc.VectorSubcoreMesh(core_axis_name="c", subcore_axis_name="s")
    fn = pl.kernel(
        _sc_body,
        out_type=[
            jax.ShapeDtypeStruct((NC, NPAD, HALF), jnp.float32),
            jax.ShapeDtypeStruct((NC, NPAD), jnp.float32),
        ],
        mesh=mesh,
        scratch_types=[
            pltpu.VMEM((CH, CK), jnp.int32),
            pltpu.VMEM((EPT,), jnp.int32),
            pltpu.VMEM((2, CK, HALF), jnp.float32),
            pltpu.VMEM((CK,), jnp.float32),
            pltpu.VMEM_SHARED((NPAD, HALF), jnp.float32),
            pltpu.VMEM_SHARED((NPAD,), jnp.float32),
            pltpu.SemaphoreType.DMA((2,)),
        ],
        name="sage_sc_aggregate",
    )
    return fn(xcat, row_r, col2, zf, zc, ones_h)


def _tc_body(x_ref, agg_ref, ca_ref, cb_ref, wx_ref, wa_ref, wb_ref, b_ref,
             o_ref):
    cinv = 1.0 / (ca_ref[...] + cb_ref[...] + 1e-6)
    y = jnp.dot(x_ref[...], wx_ref[...], preferred_element_type=jnp.float32)
    y = y + jnp.dot(agg_ref[0] * cinv, wa_ref[...],
                    preferred_element_type=jnp.float32)
    y = y + jnp.dot(agg_ref[1] * cinv, wb_ref[...],
                    preferred_element_type=jnp.float32)
    o_ref[...] = jnp.maximum(y + b_ref[...], 0.0)


def _tc_fuse(x, agg2, ca, cb, wx, wa, wb, b2):
    R = 400
    grid = (N // R,)
    return pl.pallas_call(
        _tc_body,
        grid=grid,
        in_specs=[
            pl.BlockSpec((R, D), lambda i: (i, 0)),
            pl.BlockSpec((NC, R, HALF), lambda i: (0, i, 0)),
            pl.BlockSpec((R, 1), lambda i: (i, 0)),
            pl.BlockSpec((R, 1), lambda i: (i, 0)),
            pl.BlockSpec((D, D), lambda i: (0, 0)),
            pl.BlockSpec((HALF, D), lambda i: (0, 0)),
            pl.BlockSpec((HALF, D), lambda i: (0, 0)),
            pl.BlockSpec((1, D), lambda i: (0, 0)),
        ],
        out_specs=pl.BlockSpec((R, D), lambda i: (i, 0)),
        out_shape=jax.ShapeDtypeStruct((N, D), jnp.float32),
        name="sage_tc_fuse",
    )(x, agg2, ca, cb, wx, wa, wb, b2)


def kernel(x, edge_index, W, b):
    row = edge_index[0].astype(jnp.int32)
    col = edge_index[1].astype(jnp.int32)
    row_r = row.reshape(NS, CH, CK)
    col_r = col.reshape(NS, EPT)
    col2 = jnp.stack([2 * col_r, 2 * col_r + 1])
    xcat = x.reshape(2 * N, HALF)
    zf = jnp.zeros((RPT, HALF), jnp.float32)
    zc = jnp.zeros((RPT,), jnp.float32)
    ones_h = jnp.ones((CK,), jnp.float32)

    agg2, cpart = _sc_aggregate(xcat, row_r, col2, zf, zc, ones_h)

    ca = cpart[0][:N].reshape(N, 1)
    cb = cpart[1][:N].reshape(N, 1)
    Wt = W.T
    wx = Wt[:D]
    wa = Wt[D:D + HALF]
    wb = Wt[D + HALF:]
    return _tc_fuse(x, agg2, ca, cb, wx, wa, wb, b.reshape(1, D))


# speedup vs baseline: 2.0322x; 1.0767x over previous
import jax
import jax.numpy as jnp
from jax import lax
from jax.experimental import pallas as pl
from jax.experimental.pallas import tpu as pltpu
from jax.experimental.pallas import tpu_sc as plsc

N = 10000
E = 160000
D = 256
HALF = D // 2
NS = 16
NC = 2
EPT = E // NS
CK = 80
CH = EPT // CK
NPAD = 10240
RPT = NPAD // NS


def _sc_body(xcat, row_r, col_r, zf, zc, ones_h,
             agg_out, cnt_out,
             rowv, colv, gbuf, onesv, s_agg, s_cnt, sem):
    cid = lax.axis_index("c")
    sid = lax.axis_index("s")

    pltpu.sync_copy(zf, s_agg.at[pl.ds(sid * RPT, RPT)])
    pltpu.sync_copy(zc, s_cnt.at[pl.ds(sid * RPT, RPT)])

    pltpu.sync_copy(row_r.at[sid], rowv)
    pltpu.sync_copy(col_r.at[sid], colv)
    pltpu.sync_copy(ones_h, onesv)

    plsc.subcore_barrier()

    def gather_start(j, p):
        pltpu.async_copy(
            xcat.at[cid].at[colv.at[pl.ds(j * CK, CK)]], gbuf.at[p],
            sem.at[p])

    def gather_wait(p):
        pltpu.make_async_copy(zf.at[pl.ds(0, CK)], gbuf.at[p],
                              sem.at[p]).wait()

    gather_start(0, 0)
    gather_start(1, 1)

    def step(j, carry):
        p = lax.rem(j, 2)
        gather_wait(p)

        pltpu.sync_copy(gbuf.at[p], s_agg.at[rowv.at[j]], add=True)

        @pl.when(p == cid)
        def _():
            pltpu.sync_copy(onesv, s_cnt.at[rowv.at[j]], add=True)

        @pl.when(j < CH - 2)
        def _():
            gather_start(j + 2, p)
        return carry

    lax.fori_loop(0, CH, step, 0)

    plsc.subcore_barrier()

    pltpu.sync_copy(s_agg.at[pl.ds(sid * RPT, RPT)],
                    agg_out.at[cid, pl.ds(sid * RPT, RPT)])
    pltpu.sync_copy(s_cnt.at[pl.ds(sid * RPT, RPT)],
                    cnt_out.at[cid, pl.ds(sid * RPT, RPT)])


def _sc_aggregate(xcat, row_r, col_r, zf, zc, ones_h):
    mesh = plsc.VectorSubcoreMesh(core_axis_name="c", subcore_axis_name="s")
    fn = pl.kernel(
        _sc_body,
        out_type=[
            jax.ShapeDtypeStruct((NC, NPAD, HALF), jnp.float32),
            jax.ShapeDtypeStruct((NC, NPAD), jnp.float32),
        ],
        mesh=mesh,
        scratch_types=[
            pltpu.VMEM((CH, CK), jnp.int32),
            pltpu.VMEM((EPT,), jnp.int32),
            pltpu.VMEM((2, CK, HALF), jnp.float32),
            pltpu.VMEM((CK,), jnp.float32),
            pltpu.VMEM_SHARED((NPAD, HALF), jnp.float32),
            pltpu.VMEM_SHARED((NPAD,), jnp.float32),
            pltpu.SemaphoreType.DMA((2,)),
        ],
        name="sage_sc_aggregate",
    )
    return fn(xcat, row_r, col_r, zf, zc, ones_h)


def _tc_body(x_ref, agg_ref, ca_ref, cb_ref, wx_ref, wa_ref, wb_ref, b_ref,
             o_ref):
    cinv = 1.0 / (ca_ref[...] + cb_ref[...] + 1e-6)
    y = jnp.dot(x_ref[...], wx_ref[...], preferred_element_type=jnp.float32)
    y = y + jnp.dot(agg_ref[0] * cinv, wa_ref[...],
                    preferred_element_type=jnp.float32)
    y = y + jnp.dot(agg_ref[1] * cinv, wb_ref[...],
                    preferred_element_type=jnp.float32)
    o_ref[...] = jnp.maximum(y + b_ref[...], 0.0)


def _tc_fuse(x, agg2, ca, cb, wx, wa, wb, b2):
    R = 2000
    grid = (N // R,)
    return pl.pallas_call(
        _tc_body,
        grid=grid,
        in_specs=[
            pl.BlockSpec((R, D), lambda i: (i, 0)),
            pl.BlockSpec((NC, R, HALF), lambda i: (0, i, 0)),
            pl.BlockSpec((R, 1), lambda i: (i, 0)),
            pl.BlockSpec((R, 1), lambda i: (i, 0)),
            pl.BlockSpec((D, D), lambda i: (0, 0)),
            pl.BlockSpec((HALF, D), lambda i: (0, 0)),
            pl.BlockSpec((HALF, D), lambda i: (0, 0)),
            pl.BlockSpec((1, D), lambda i: (0, 0)),
        ],
        out_specs=pl.BlockSpec((R, D), lambda i: (i, 0)),
        out_shape=jax.ShapeDtypeStruct((N, D), jnp.float32),
        name="sage_tc_fuse",
    )(x, agg2, ca, cb, wx, wa, wb, b2)


def kernel(x, edge_index, W, b):
    row = edge_index[0].astype(jnp.int32)
    col = edge_index[1].astype(jnp.int32)
    row_r = row.reshape(NS, CH, CK)
    col_r = col.reshape(NS, EPT)
    xcat = jnp.stack([x[:, :HALF], x[:, HALF:]])
    zf = jnp.zeros((RPT, HALF), jnp.float32)
    zc = jnp.zeros((RPT,), jnp.float32)
    ones_h = jnp.ones((CK,), jnp.float32)

    agg2, cpart = _sc_aggregate(xcat, row_r, col_r, zf, zc, ones_h)

    ca = cpart[0][:N].reshape(N, 1)
    cb = cpart[1][:N].reshape(N, 1)
    Wt = W.T
    wx = Wt[:D]
    wa = Wt[D:D + HALF]
    wb = Wt[D + HALF:]
    return _tc_fuse(x, agg2, ca, cb, wx, wa, wb, b.reshape(1, D))


# speedup vs baseline: 2.3872x; 1.1746x over previous
import jax
import jax.numpy as jnp
from jax import lax
from jax.experimental import pallas as pl
from jax.experimental.pallas import tpu as pltpu
from jax.experimental.pallas import tpu_sc as plsc

N = 10000
E = 160000
D = 256
HALF = D // 2
NS = 16
NC = 2
EPT = E // NS
CK = 80
CH = EPT // CK
NPAD = 10240
NBUF = 3
RPT = NPAD // NS


def _sc_body(xcat, row_r, col_r, zf, zc, ones_h,
             agg_out, cnt_out,
             rring, colv, gbuf, onesv, s_agg, s_cnt, sem, rsem):
    cid = lax.axis_index("c")
    sid = lax.axis_index("s")

    pltpu.sync_copy(zf, s_agg.at[pl.ds(sid * RPT, RPT)])
    pltpu.sync_copy(zc, s_cnt.at[pl.ds(sid * RPT, RPT)])

    pltpu.sync_copy(col_r.at[sid], colv)
    pltpu.sync_copy(ones_h, onesv)

    plsc.subcore_barrier()

    def fetch_start(j, p):
        pltpu.async_copy(
            xcat.at[cid].at[colv.at[pl.ds(j * CK, CK)]], gbuf.at[p],
            sem.at[p])
        pltpu.async_copy(row_r.at[sid, j], rring.at[p], rsem.at[p])

    def fetch_wait(p):
        pltpu.make_async_copy(zf.at[pl.ds(0, CK)], gbuf.at[p],
                              sem.at[p]).wait()
        pltpu.make_async_copy(row_r.at[0, 0], rring.at[p],
                              rsem.at[p]).wait()

    for b in range(NBUF):
        fetch_start(b, b)

    def step(j, carry):
        p = lax.rem(j, NBUF)
        fetch_wait(p)

        pltpu.sync_copy(gbuf.at[p], s_agg.at[rring.at[p]], add=True)

        @pl.when(lax.rem(j, 2) == cid)
        def _():
            pltpu.sync_copy(onesv, s_cnt.at[rring.at[p]], add=True)

        @pl.when(j < CH - NBUF)
        def _():
            fetch_start(j + NBUF, p)
        return carry

    lax.fori_loop(0, CH, step, 0)

    plsc.subcore_barrier()

    pltpu.sync_copy(s_agg.at[pl.ds(sid * RPT, RPT)],
                    agg_out.at[cid, pl.ds(sid * RPT, RPT)])
    pltpu.sync_copy(s_cnt.at[pl.ds(sid * RPT, RPT)],
                    cnt_out.at[cid, pl.ds(sid * RPT, RPT)])


def _sc_aggregate(xcat, row_r, col_r, zf, zc, ones_h):
    mesh = plsc.VectorSubcoreMesh(core_axis_name="c", subcore_axis_name="s")
    fn = pl.kernel(
        _sc_body,
        out_type=[
            jax.ShapeDtypeStruct((NC, NPAD, HALF), jnp.float32),
            jax.ShapeDtypeStruct((NC, NPAD), jnp.float32),
        ],
        mesh=mesh,
        scratch_types=[
            pltpu.VMEM((NBUF, CK), jnp.int32),
            pltpu.VMEM((EPT,), jnp.int32),
            pltpu.VMEM((NBUF, CK, HALF), jnp.float32),
            pltpu.VMEM((CK,), jnp.float32),
            pltpu.VMEM_SHARED((NPAD, HALF), jnp.float32),
            pltpu.VMEM_SHARED((NPAD,), jnp.float32),
            pltpu.SemaphoreType.DMA((NBUF,)),
            pltpu.SemaphoreType.DMA((NBUF,)),
        ],
        name="sage_sc_aggregate",
    )
    return fn(xcat, row_r, col_r, zf, zc, ones_h)


def _tc_body(x_ref, agg_ref, ca_ref, cb_ref, wx_ref, wa_ref, wb_ref, b_ref,
             o_ref):
    cinv = 1.0 / (ca_ref[...] + cb_ref[...] + 1e-6)
    y = jnp.dot(x_ref[...], wx_ref[...], preferred_element_type=jnp.float32)
    y = y + jnp.dot(agg_ref[0] * cinv, wa_ref[...],
                    preferred_element_type=jnp.float32)
    y = y + jnp.dot(agg_ref[1] * cinv, wb_ref[...],
                    preferred_element_type=jnp.float32)
    o_ref[...] = jnp.maximum(y + b_ref[...], 0.0)


def _tc_fuse(x, agg2, ca, cb, wx, wa, wb, b2):
    R = 2000
    grid = (N // R,)
    return pl.pallas_call(
        _tc_body,
        grid=grid,
        in_specs=[
            pl.BlockSpec((R, D), lambda i: (i, 0)),
            pl.BlockSpec((NC, R, HALF), lambda i: (0, i, 0)),
            pl.BlockSpec((R, 1), lambda i: (i, 0)),
            pl.BlockSpec((R, 1), lambda i: (i, 0)),
            pl.BlockSpec((D, D), lambda i: (0, 0)),
            pl.BlockSpec((HALF, D), lambda i: (0, 0)),
            pl.BlockSpec((HALF, D), lambda i: (0, 0)),
            pl.BlockSpec((1, D), lambda i: (0, 0)),
        ],
        out_specs=pl.BlockSpec((R, D), lambda i: (i, 0)),
        out_shape=jax.ShapeDtypeStruct((N, D), jnp.float32),
        name="sage_tc_fuse",
    )(x, agg2, ca, cb, wx, wa, wb, b2)


def kernel(x, edge_index, W, b):
    row = edge_index[0].astype(jnp.int32)
    col = edge_index[1].astype(jnp.int32)
    row_r = row.reshape(NS, CH, CK)
    col_r = col.reshape(NS, EPT)
    xcat = jnp.stack([x[:, :HALF], x[:, HALF:]])
    zf = jnp.zeros((RPT, HALF), jnp.float32)
    zc = jnp.zeros((RPT,), jnp.float32)
    ones_h = jnp.ones((CK,), jnp.float32)

    agg2, cpart = _sc_aggregate(xcat, row_r, col_r, zf, zc, ones_h)

    ca = cpart[0][:N].reshape(N, 1)
    cb = cpart[1][:N].reshape(N, 1)
    Wt = W.T
    wx = Wt[:D]
    wa = Wt[D:D + HALF]
    wb = Wt[D + HALF:]
    return _tc_fuse(x, agg2, ca, cb, wx, wa, wb, b.reshape(1, D))
